# baseline (speedup 1.0000x reference)
import functools

import jax
import jax.numpy as jnp
from jax import lax
from jax.experimental import pallas as pl
from jax.experimental.pallas import tpu as pltpu
from jax.experimental.pallas import tpu_sc as plsc

_HASH_VOCAB = 1000000
_BIGRAM_DIM = 64
_MODEL_DIM = 1024
_BOS_ID = 1

_NC = 2
_NS = 16
_NW = _NC * _NS
_LANES = 16

_B = 4
_S = 8192
_TOTAL = _B * _S
_CHUNK = _TOTAL // _NW
_IDXW = 128
_NIDX = _CHUNK // _IDXW


def _sc_hash_gather_body(ids_hbm, embed_hbm, out_hbm, ids_ext, hash_v, rows_v, sem):
    wid = lax.axis_index("s") * _NC + lax.axis_index("c")
    base = wid * _CHUNK

    ids_ext[pl.ds(0, _LANES)] = jnp.full((_LANES,), _BOS_ID, dtype=jnp.int32)
    pltpu.sync_copy(ids_hbm.at[pl.ds(base, _CHUNK)], ids_ext.at[pl.ds(8, _CHUNK)])

    @pl.when(lax.rem(wid, _S // _CHUNK) != 0)
    def _():
        pltpu.sync_copy(ids_hbm.at[pl.ds(base - 8, 8)], ids_ext.at[pl.ds(0, 8)])

    lanes = lax.iota(jnp.int32, _LANES)
    for j in range(_NIDX):
        for k in range(_IDXW // _LANES):
            i = j * (_IDXW // _LANES) + k
            cur = ids_ext[pl.ds(8 + i * _LANES, _LANES)]
            prev = plsc.load_gather(ids_ext, [lanes + (7 + i * _LANES)])
            x = (prev << 8) + (prev << 2) + (prev << 1) + prev + cur
            for c in (8000000, 4000000, 2000000, 1000000):
                x = jnp.where(x >= c, x - c, x)
            hash_v[j, pl.ds(k * _LANES, _LANES)] = x

    copies = []
    for j in range(_NIDX):
        copies.append(
            pltpu.async_copy(
                embed_hbm.at[hash_v.at[j]],
                rows_v.at[pl.ds(j * _IDXW, _IDXW)],
                sem,
            )
        )
    for c in copies:
        c.wait()

    pltpu.sync_copy(rows_v, out_hbm.at[pl.ds(base, _CHUNK)])


def _sc_hash_gather(ids_flat, embed):
    mesh = plsc.VectorSubcoreMesh(core_axis_name="c", subcore_axis_name="s")
    return pl.kernel(
        _sc_hash_gather_body,
        out_type=jax.ShapeDtypeStruct((_TOTAL, _BIGRAM_DIM), jnp.float32),
        mesh=mesh,
        scratch_types=[
            pltpu.VMEM((_CHUNK + 8,), jnp.int32),
            pltpu.VMEM((_NIDX, _IDXW), jnp.int32),
            pltpu.VMEM((_CHUNK, _BIGRAM_DIM), jnp.float32),
            pltpu.SemaphoreType.DMA,
        ],
        compiler_params=pltpu.CompilerParams(
            needs_layout_passes=False, use_tc_tiling_on_sc=False
        ),
    )(ids_flat, embed)


_MM_ROWS = 2048


def _mm_body(emb_ref, w_ref, out_ref):
    out_ref[...] = jnp.dot(
        emb_ref[...], w_ref[...], preferred_element_type=jnp.float32
    )


def _tc_matmul(emb, W):
    return pl.pallas_call(
        _mm_body,
        grid=(_TOTAL // _MM_ROWS,),
        in_specs=[
            pl.BlockSpec((_MM_ROWS, _BIGRAM_DIM), lambda i: (i, 0)),
            pl.BlockSpec((_BIGRAM_DIM, _MODEL_DIM), lambda i: (0, 0)),
        ],
        out_specs=pl.BlockSpec((_MM_ROWS, _MODEL_DIM), lambda i: (i, 0)),
        out_shape=jax.ShapeDtypeStruct((_TOTAL, _MODEL_DIM), jnp.float32),
    )(emb, W)


def kernel(ids, embed, W):
    ids_flat = ids.reshape(_TOTAL).astype(jnp.int32)
    emb = _sc_hash_gather(ids_flat, embed)
    out = _tc_matmul(emb, W)
    return out.reshape(_B, _S, _MODEL_DIM)
